# Initial kernel scaffold; baseline (speedup 1.0000x reference)
import jax
import jax.numpy as jnp
from jax.experimental import pallas as pl


def kernel(x, embed):
    raise NotImplementedError("write your pallas kernel here")


# speedup vs baseline: 1.3906x; 1.3906x over previous
import functools

import jax
import jax.numpy as jnp
from jax import lax
from jax.experimental import pallas as pl
from jax.experimental.pallas import tpu as pltpu
from jax.experimental.pallas import tpu_sc as plsc


_ROWS_PER_BLOCK = 512


def _argmin_body(x2_ref, xb_ref, emb_ref, e2_ref, idx_ref):
    m = lax.dot_general(
        xb_ref[...].astype(jnp.bfloat16), emb_ref[...],
        dimension_numbers=(((1,), (1,)), ((), ())),
        preferred_element_type=jnp.float32,
    )
    d = (x2_ref[...] - 2.0 * m) + e2_ref[...]
    rmin = jnp.min(d, axis=1, keepdims=True)
    cols = lax.broadcasted_iota(jnp.int32, d.shape, 1)
    idx = jnp.min(jnp.where(d == rmin, cols, jnp.int32(2**30)),
                  axis=1, keepdims=True)
    idx_ref[...] = idx


def _tc_argmin(x2, xf, embed, e2):
    n, dim = xf.shape
    k = embed.shape[0]
    r = _ROWS_PER_BLOCK
    grid = (n // r,)
    return pl.pallas_call(
        _argmin_body,
        grid=grid,
        in_specs=[
            pl.BlockSpec((r, 1), lambda i: (i, 0)),
            pl.BlockSpec((r, dim), lambda i: (i, 0)),
            pl.BlockSpec((k, dim), lambda i: (0, 0)),
            pl.BlockSpec((1, k), lambda i: (0, 0)),
        ],
        out_specs=pl.BlockSpec((r, 1), lambda i: (i, 0)),
        out_shape=jax.ShapeDtypeStruct((n, 1), jnp.int32),
    )(x2, xf, embed, e2)


@functools.lru_cache(maxsize=None)
def _make_sc_gather(v, d, b):
    info = plsc.get_sparse_core_info()
    nc, ns = info.num_cores, info.num_subcores
    nw = nc * ns
    assert b % (8 * nw) == 0 and d % 128 == 0
    b_per_w = b // nw
    mesh = plsc.VectorSubcoreMesh(core_axis_name="c", subcore_axis_name="s")

    @functools.partial(
        pl.kernel, mesh=mesh,
        out_type=jax.ShapeDtypeStruct((b, d), jnp.float32),
        scratch_types=[
            pltpu.VMEM((b_per_w,), jnp.int32),
            pltpu.VMEM((b_per_w, d), jnp.float32),
            pltpu.SemaphoreType.DMA,
        ],
    )
    def gather_kernel(table_hbm, idx_hbm, out_hbm, idx_v, rows_v, sem):
        wid = lax.axis_index("s") * nc + lax.axis_index("c")
        base = wid * b_per_w
        pltpu.sync_copy(idx_hbm.at[pl.ds(base, b_per_w)], idx_v)
        pltpu.async_copy(table_hbm.at[idx_v], rows_v, sem).wait()
        pltpu.sync_copy(rows_v, out_hbm.at[pl.ds(base, b_per_w)])

    return gather_kernel


def kernel(x, embed):
    shape = x.shape
    xf = x.reshape(-1, shape[-1])
    embed_t = embed.T
    x2 = jnp.sum(xf ** 2, axis=1, keepdims=True)
    e2 = jnp.sum(embed_t ** 2, axis=0, keepdims=True)
    idx = _tc_argmin(x2, xf, embed, e2).reshape(-1)
    dim = embed.shape[1]
    table = jnp.pad(embed, ((0, 0), (0, 128 - dim)))
    rows = _make_sc_gather(embed.shape[0], 128, idx.shape[0])(table, idx)
    quantize = rows[:, :dim]
    return quantize.reshape(shape), idx.reshape(shape[:-1])


# speedup vs baseline: 1.4702x; 1.0572x over previous
import functools

import jax
import jax.numpy as jnp
from jax import lax
from jax.experimental import pallas as pl
from jax.experimental.pallas import tpu as pltpu
from jax.experimental.pallas import tpu_sc as plsc


_ROWS_PER_BLOCK = 512


def _argmin_body(x2_ref, xb_ref, emb_ref, e2_ref, idx_ref):
    m = lax.dot_general(
        xb_ref[...].astype(jnp.bfloat16), emb_ref[...],
        dimension_numbers=(((1,), (1,)), ((), ())),
        preferred_element_type=jnp.float32,
    )
    d = (x2_ref[...] - 2.0 * m) + e2_ref[...]
    idx_ref[...] = jnp.argmin(d, axis=1, keepdims=True).astype(jnp.int32)


def _tc_argmin(x2, xf, embed, e2):
    n, dim = xf.shape
    k = embed.shape[0]
    r = _ROWS_PER_BLOCK
    grid = (n // r,)
    return pl.pallas_call(
        _argmin_body,
        grid=grid,
        in_specs=[
            pl.BlockSpec((r, 1), lambda i: (i, 0)),
            pl.BlockSpec((r, dim), lambda i: (i, 0)),
            pl.BlockSpec((k, dim), lambda i: (0, 0)),
            pl.BlockSpec((1, k), lambda i: (0, 0)),
        ],
        out_specs=pl.BlockSpec((r, 1), lambda i: (i, 0)),
        out_shape=jax.ShapeDtypeStruct((n, 1), jnp.int32),
    )(x2, xf, embed, e2)


@functools.lru_cache(maxsize=None)
def _make_sc_gather(v, d, b):
    info = plsc.get_sparse_core_info()
    nc, ns = info.num_cores, info.num_subcores
    nw = nc * ns
    assert b % (8 * nw) == 0 and d % 128 == 0
    b_per_w = b // nw
    mesh = plsc.VectorSubcoreMesh(core_axis_name="c", subcore_axis_name="s")

    @functools.partial(
        pl.kernel, mesh=mesh,
        out_type=jax.ShapeDtypeStruct((b, d), jnp.float32),
        scratch_types=[
            pltpu.VMEM((b_per_w,), jnp.int32),
            pltpu.VMEM((b_per_w, d), jnp.float32),
            pltpu.SemaphoreType.DMA,
        ],
    )
    def gather_kernel(table_hbm, idx_hbm, out_hbm, idx_v, rows_v, sem):
        wid = lax.axis_index("s") * nc + lax.axis_index("c")
        base = wid * b_per_w
        pltpu.sync_copy(idx_hbm.at[pl.ds(base, b_per_w)], idx_v)
        pltpu.async_copy(table_hbm.at[idx_v], rows_v, sem).wait()
        pltpu.sync_copy(rows_v, out_hbm.at[pl.ds(base, b_per_w)])

    return gather_kernel


def kernel(x, embed):
    shape = x.shape
    xf = x.reshape(-1, shape[-1])
    embed_t = embed.T
    x2 = jnp.sum(xf ** 2, axis=1, keepdims=True)
    e2 = jnp.sum(embed_t ** 2, axis=0, keepdims=True)
    idx = _tc_argmin(x2, xf, embed, e2).reshape(-1)
    dim = embed.shape[1]
    table = jnp.pad(embed, ((0, 0), (0, 128 - dim)))
    rows = _make_sc_gather(embed.shape[0], 128, idx.shape[0])(table, idx)
    quantize = rows[:, :dim]
    return quantize.reshape(shape), idx.reshape(shape[:-1])


# speedup vs baseline: 1.5488x; 1.0535x over previous
import functools

import jax
import jax.numpy as jnp
from jax import lax
from jax.experimental import pallas as pl
from jax.experimental.pallas import tpu as pltpu
from jax.experimental.pallas import tpu_sc as plsc


_ROWS_PER_BLOCK = 1024


def _argmin_body(x2_ref, xb_ref, emb_ref, e2_ref, idx_ref):
    m = lax.dot_general(
        xb_ref[...].astype(jnp.bfloat16), emb_ref[...],
        dimension_numbers=(((1,), (1,)), ((), ())),
        preferred_element_type=jnp.float32,
    )
    d = (x2_ref[...] - 2.0 * m) + e2_ref[...]
    idx_ref[...] = jnp.argmin(d, axis=1, keepdims=True).astype(jnp.int32)


def _tc_argmin(x2, xf, embed, e2):
    n, dim = xf.shape
    k = embed.shape[0]
    r = _ROWS_PER_BLOCK
    grid = (n // r,)
    return pl.pallas_call(
        _argmin_body,
        grid=grid,
        in_specs=[
            pl.BlockSpec((r, 1), lambda i: (i, 0)),
            pl.BlockSpec((r, dim), lambda i: (i, 0)),
            pl.BlockSpec((k, dim), lambda i: (0, 0)),
            pl.BlockSpec((1, k), lambda i: (0, 0)),
        ],
        out_specs=pl.BlockSpec((r, 1), lambda i: (i, 0)),
        out_shape=jax.ShapeDtypeStruct((n, 1), jnp.int32),
    )(x2, xf, embed, e2)


@functools.lru_cache(maxsize=None)
def _make_sc_gather(v, d, b):
    info = plsc.get_sparse_core_info()
    nc, ns = info.num_cores, info.num_subcores
    nw = nc * ns
    assert b % (8 * nw) == 0 and d % 128 == 0
    b_per_w = b // nw
    mesh = plsc.VectorSubcoreMesh(core_axis_name="c", subcore_axis_name="s")

    @functools.partial(
        pl.kernel, mesh=mesh,
        out_type=jax.ShapeDtypeStruct((b, d), jnp.float32),
        scratch_types=[
            pltpu.VMEM((b_per_w,), jnp.int32),
            pltpu.VMEM((b_per_w, d), jnp.float32),
            pltpu.SemaphoreType.DMA,
        ],
    )
    def gather_kernel(table_hbm, idx_hbm, out_hbm, idx_v, rows_v, sem):
        wid = lax.axis_index("s") * nc + lax.axis_index("c")
        base = wid * b_per_w
        pltpu.sync_copy(idx_hbm.at[pl.ds(base, b_per_w)], idx_v)
        pltpu.async_copy(table_hbm.at[idx_v], rows_v, sem).wait()
        pltpu.sync_copy(rows_v, out_hbm.at[pl.ds(base, b_per_w)])

    return gather_kernel


def kernel(x, embed):
    shape = x.shape
    xf = x.reshape(-1, shape[-1])
    embed_t = embed.T
    x2 = jnp.sum(xf ** 2, axis=1, keepdims=True)
    e2 = jnp.sum(embed_t ** 2, axis=0, keepdims=True)
    idx = _tc_argmin(x2, xf, embed, e2).reshape(-1)
    dim = embed.shape[1]
    table = jnp.pad(embed, ((0, 0), (0, 128 - dim)))
    rows = _make_sc_gather(embed.shape[0], 128, idx.shape[0])(table, idx)
    quantize = rows[:, :dim]
    return quantize.reshape(shape), idx.reshape(shape[:-1])


# speedup vs baseline: 1.5866x; 1.0244x over previous
import functools

import jax
import jax.numpy as jnp
from jax import lax
from jax.experimental import pallas as pl
from jax.experimental.pallas import tpu as pltpu
from jax.experimental.pallas import tpu_sc as plsc


_ROWS_PER_BLOCK = 2048


def _argmin_body(x2_ref, xb_ref, emb_ref, e2_ref, idx_ref):
    m = lax.dot_general(
        xb_ref[...].astype(jnp.bfloat16), emb_ref[...],
        dimension_numbers=(((1,), (1,)), ((), ())),
        preferred_element_type=jnp.float32,
    )
    d = (x2_ref[...] - 2.0 * m) + e2_ref[...]
    idx_ref[...] = jnp.argmin(d, axis=1, keepdims=True).astype(jnp.int32)


def _tc_argmin(x2, xf, embed, e2):
    n, dim = xf.shape
    k = embed.shape[0]
    r = _ROWS_PER_BLOCK
    grid = (n // r,)
    return pl.pallas_call(
        _argmin_body,
        grid=grid,
        in_specs=[
            pl.BlockSpec((r, 1), lambda i: (i, 0)),
            pl.BlockSpec((r, dim), lambda i: (i, 0)),
            pl.BlockSpec((k, dim), lambda i: (0, 0)),
            pl.BlockSpec((1, k), lambda i: (0, 0)),
        ],
        out_specs=pl.BlockSpec((r, 1), lambda i: (i, 0)),
        out_shape=jax.ShapeDtypeStruct((n, 1), jnp.int32),
    )(x2, xf, embed, e2)


@functools.lru_cache(maxsize=None)
def _make_sc_gather(v, d, b):
    info = plsc.get_sparse_core_info()
    nc, ns = info.num_cores, info.num_subcores
    nw = nc * ns
    assert b % (8 * nw) == 0 and d % 128 == 0
    b_per_w = b // nw
    mesh = plsc.VectorSubcoreMesh(core_axis_name="c", subcore_axis_name="s")

    @functools.partial(
        pl.kernel, mesh=mesh,
        out_type=jax.ShapeDtypeStruct((b, d), jnp.float32),
        scratch_types=[
            pltpu.VMEM((b_per_w,), jnp.int32),
            pltpu.VMEM((b_per_w, d), jnp.float32),
            pltpu.SemaphoreType.DMA,
        ],
    )
    def gather_kernel(table_hbm, idx_hbm, out_hbm, idx_v, rows_v, sem):
        wid = lax.axis_index("s") * nc + lax.axis_index("c")
        base = wid * b_per_w
        pltpu.sync_copy(idx_hbm.at[pl.ds(base, b_per_w)], idx_v)
        pltpu.async_copy(table_hbm.at[idx_v], rows_v, sem).wait()
        pltpu.sync_copy(rows_v, out_hbm.at[pl.ds(base, b_per_w)])

    return gather_kernel


def kernel(x, embed):
    shape = x.shape
    xf = x.reshape(-1, shape[-1])
    embed_t = embed.T
    x2 = jnp.sum(xf ** 2, axis=1, keepdims=True)
    e2 = jnp.sum(embed_t ** 2, axis=0, keepdims=True)
    idx = _tc_argmin(x2, xf, embed, e2).reshape(-1)
    dim = embed.shape[1]
    table = jnp.pad(embed, ((0, 0), (0, 128 - dim)))
    rows = _make_sc_gather(embed.shape[0], 128, idx.shape[0])(table, idx)
    quantize = rows[:, :dim]
    return quantize.reshape(shape), idx.reshape(shape[:-1])
